# Initial kernel scaffold; baseline (speedup 1.0000x reference)
import jax
import jax.numpy as jnp
from jax.experimental import pallas as pl


def kernel(x, edge_index, W_self1, W_neigh1, b1, W_self2, W_neigh2, b2):
    raise NotImplementedError("write your pallas kernel here")


# speedup vs baseline: 9.8913x; 9.8913x over previous
import functools

import jax
import jax.numpy as jnp
from jax import lax
from jax.experimental import pallas as pl
from jax.experimental.pallas import tpu as pltpu
from jax.experimental.pallas import tpu_sc as plsc

N = 10000
E = 320000
D = 128
L = 16
NC = 2
NS = 16
NW = NC * NS
NPAD = 10240
RPS = NPAD // NS
EROWS = 2560
RPW = EROWS // NW
EPAD = EROWS * 128 - E

_mesh = plsc.VectorSubcoreMesh(core_axis_name="c", subcore_axis_name="s")
_sc_params = pltpu.CompilerParams(use_tc_tiling_on_sc=False)


def _fill_2d(ref, nrows, ncols, value):
    vec = jnp.full((L,), value, jnp.float32)

    def body(i, carry):
        for cband in range(ncols // L):
            ref[i, pl.ds(cband * L, L)] = vec
        return carry

    lax.fori_loop(0, nrows, body, 0)


@functools.partial(
    pl.kernel,
    mesh=_mesh,
    out_type=(
        jax.ShapeDtypeStruct((NC, NPAD, D), jnp.float32),
        jax.ShapeDtypeStruct((NC, NPAD, L), jnp.float32),
    ),
    scratch_types=[
        pltpu.VMEM((16, 128), jnp.int32),
        pltpu.VMEM((16, 128), jnp.int32),
        pltpu.VMEM((128, D), jnp.float32),
        pltpu.VMEM((128, L), jnp.float32),
        pltpu.VMEM((128, L), jnp.float32),
        pltpu.VMEM_SHARED((NPAD, D), jnp.float32),
        pltpu.VMEM_SHARED((NPAD, L), jnp.float32),
    ],
    compiler_params=_sc_params,
)
def _sc_layer1_agg(x_hbm, src_hbm, dst_hbm, p_out, deg_out,
                   src_v, dst_v, rows_v, ones_v, zd_v, acc_sh, deg_sh):
    c = lax.axis_index("c")
    s = lax.axis_index("s")
    wid = s * NC + c
    r0 = wid * RPW

    _fill_2d(rows_v, 128, D, 0.0)
    _fill_2d(zd_v, 128, L, 0.0)
    _fill_2d(ones_v, 128, L, 1.0)
    for k in range(RPS // 128):
        pltpu.sync_copy(rows_v, acc_sh.at[pl.ds(s * RPS + k * 128, 128)])
        pltpu.sync_copy(zd_v, deg_sh.at[pl.ds(s * RPS + k * 128, 128)])
    plsc.subcore_barrier()

    def body(j, carry):
        pltpu.sync_copy(x_hbm.at[src_v.at[j]], rows_v)
        pltpu.sync_copy(rows_v, acc_sh.at[dst_v.at[j]], add=True)
        pltpu.sync_copy(ones_v, deg_sh.at[dst_v.at[j]], add=True)
        return carry

    for stage in range(RPW // 16):
        pltpu.sync_copy(src_hbm.at[pl.ds(r0 + stage * 16, 16)], src_v)
        pltpu.sync_copy(dst_hbm.at[pl.ds(r0 + stage * 16, 16)], dst_v)
        lax.fori_loop(0, 16, body, 0)
    plsc.subcore_barrier()

    for k in range(RPS // 128):
        sl = pl.ds(s * RPS + k * 128, 128)
        pltpu.sync_copy(acc_sh.at[sl], rows_v)
        pltpu.sync_copy(rows_v, p_out.at[c, sl])
        pltpu.sync_copy(deg_sh.at[sl], zd_v)
        pltpu.sync_copy(zd_v, deg_out.at[c, sl])


@functools.partial(
    pl.kernel,
    mesh=_mesh,
    out_type=jax.ShapeDtypeStruct((NC, NPAD, L), jnp.float32),
    scratch_types=[
        pltpu.VMEM((16, 128), jnp.int32),
        pltpu.VMEM((16, 128), jnp.int32),
        pltpu.VMEM((128, L), jnp.float32),
        pltpu.VMEM((128, L), jnp.float32),
        pltpu.VMEM_SHARED((NPAD, L), jnp.float32),
    ],
    compiler_params=_sc_params,
)
def _sc_layer2_agg(s16_hbm, src_hbm, dst_hbm, s_out,
                   src_v, dst_v, vals_v, zd_v, acc_sh):
    c = lax.axis_index("c")
    s = lax.axis_index("s")
    wid = s * NC + c
    r0 = wid * RPW

    _fill_2d(zd_v, 128, L, 0.0)
    for k in range(RPS // 128):
        pltpu.sync_copy(zd_v, acc_sh.at[pl.ds(s * RPS + k * 128, 128)])
    plsc.subcore_barrier()

    def body(j, carry):
        pltpu.sync_copy(s16_hbm.at[src_v.at[j]], vals_v)
        pltpu.sync_copy(vals_v, acc_sh.at[dst_v.at[j]], add=True)
        return carry

    for stage in range(RPW // 16):
        pltpu.sync_copy(src_hbm.at[pl.ds(r0 + stage * 16, 16)], src_v)
        pltpu.sync_copy(dst_hbm.at[pl.ds(r0 + stage * 16, 16)], dst_v)
        lax.fori_loop(0, 16, body, 0)
    plsc.subcore_barrier()

    for k in range(RPS // 128):
        sl = pl.ds(s * RPS + k * 128, 128)
        pltpu.sync_copy(acc_sh.at[sl], zd_v)
        pltpu.sync_copy(zd_v, s_out.at[c, sl])


_RB = 1024


def _tc_layer_body(x_ref, p0_ref, p1_ref, d0_ref, d1_ref,
                   ws1_ref, wn1_ref, b1_ref, ws2_ref, wn2_ref, b2_ref,
                   s_ref, t_ref):
    d = jnp.maximum(d0_ref[:, 0:1] + d1_ref[:, 0:1], 1.0)
    agg = (p0_ref[...] + p1_ref[...]) / d
    h = x_ref[...] @ ws1_ref[...] + agg @ wn1_ref[...] + b1_ref[...]
    h = jax.nn.sigmoid(h)
    s_ref[...] = jnp.broadcast_to(h @ wn2_ref[...], (_RB, L))
    t_ref[...] = h @ ws2_ref[...] + b2_ref[...]


def _tc_layer(x, p0, p1, d0, d1, ws1, wn1, b1, ws2, wn2, b2):
    grid = (NPAD // _RB,)
    row = lambda i: (i, 0)
    full = lambda i: (0, 0)
    return pl.pallas_call(
        _tc_layer_body,
        grid=grid,
        in_specs=[
            pl.BlockSpec((_RB, D), row),
            pl.BlockSpec((_RB, D), row),
            pl.BlockSpec((_RB, D), row),
            pl.BlockSpec((_RB, L), row),
            pl.BlockSpec((_RB, L), row),
            pl.BlockSpec((D, D), full),
            pl.BlockSpec((D, D), full),
            pl.BlockSpec((1, D), full),
            pl.BlockSpec((D, 1), full),
            pl.BlockSpec((D, 1), full),
            pl.BlockSpec((1, 1), full),
        ],
        out_specs=[
            pl.BlockSpec((_RB, L), row),
            pl.BlockSpec((_RB, 1), row),
        ],
        out_shape=[
            jax.ShapeDtypeStruct((NPAD, L), jnp.float32),
            jax.ShapeDtypeStruct((NPAD, 1), jnp.float32),
        ],
    )(x, p0, p1, d0, d1, ws1, wn1, b1, ws2, wn2, b2)


def _tc_final_body(t_ref, s0_ref, s1_ref, d0_ref, d1_ref, o_ref):
    d = jnp.maximum(d0_ref[:, 0:1] + d1_ref[:, 0:1], 1.0)
    o_ref[...] = t_ref[...] + (s0_ref[:, 0:1] + s1_ref[:, 0:1]) / d


def _tc_final(t, s0, s1, d0, d1):
    grid = (NPAD // _RB,)
    row = lambda i: (i, 0)
    return pl.pallas_call(
        _tc_final_body,
        grid=grid,
        in_specs=[
            pl.BlockSpec((_RB, 1), row),
            pl.BlockSpec((_RB, L), row),
            pl.BlockSpec((_RB, L), row),
            pl.BlockSpec((_RB, L), row),
            pl.BlockSpec((_RB, L), row),
        ],
        out_specs=pl.BlockSpec((_RB, 1), row),
        out_shape=jax.ShapeDtypeStruct((NPAD, 1), jnp.float32),
    )(t, s0, s1, d0, d1)


def kernel(x, edge_index, W_self1, W_neigh1, b1, W_self2, W_neigh2, b2):
    x_pad = jnp.concatenate([x, jnp.zeros((NPAD - N, D), jnp.float32)])
    pad_i = jnp.arange(EPAD, dtype=jnp.int32)
    src2d = jnp.concatenate([edge_index[0], pad_i % 128]).reshape(EROWS, 128)
    dst2d = jnp.concatenate([edge_index[1], N + pad_i % (NPAD - N)]).reshape(EROWS, 128)

    p, deg = _sc_layer1_agg(x_pad, src2d, dst2d)
    s16, t = _tc_layer(x_pad, p[0], p[1], deg[0], deg[1],
                       W_self1, W_neigh1, b1.reshape(1, D),
                       W_self2, W_neigh2, b2.reshape(1, 1))
    s2 = _sc_layer2_agg(s16, src2d, dst2d)
    out = _tc_final(t, s2[0], s2[1], deg[0], deg[1])
    return out[:N]


# speedup vs baseline: 14.7534x; 1.4915x over previous
import functools

import jax
import jax.numpy as jnp
from jax import lax
from jax.experimental import pallas as pl
from jax.experimental.pallas import tpu as pltpu
from jax.experimental.pallas import tpu_sc as plsc

N = 10000
E = 320000
D = 128
L = 16
NC = 2
NS = 16
NW = NC * NS
NPAD = 10240
RPS = NPAD // NS
EROWS = 2560
RPW = EROWS // NW
EPAD = EROWS * 128 - E
SBLK = 8
NSTG = RPW // SBLK

_mesh = plsc.VectorSubcoreMesh(core_axis_name="c", subcore_axis_name="s")
_sc_params = pltpu.CompilerParams(use_tc_tiling_on_sc=False)

def _fill_rows(ref, nrows, ncols, value):
    vec = jnp.full((L,), value, jnp.float32)

    def body(i, carry):
        for cb in range(ncols // L):
            ref[i, pl.ds(cb * L, L)] = vec
        return carry

    lax.fori_loop(0, nrows, body, 0)


def _fill_flat(ref, n, value):
    vec = jnp.full((L,), value, jnp.float32)

    def body(i, carry):
        ref[pl.ds(i * L, L)] = vec
        return carry

    lax.fori_loop(0, n // L, body, 0)


def _sc_agg_pipeline(table_hbm, src_hbm, dst_hbm, src_v, dst_v, buf_v,
                     sem_i, sem_g, sem_s, r0, acc_sh, extra=None):
    pltpu.sync_copy(src_hbm.at[pl.ds(r0, SBLK)], src_v.at[0])
    pltpu.sync_copy(dst_hbm.at[pl.ds(r0, SBLK)], dst_v.at[0])
    gather = [None] * (RPW + 1)
    scat = [None] * RPW
    degs = []
    stage_pending = []
    gather[0] = pltpu.async_copy(table_hbm.at[src_v.at[0, 0]], buf_v.at[0], sem_g)
    for j in range(RPW):
        bp = j % 2
        sp = (j // SBLK) % 2
        b = j % SBLK
        if b == 0 and j + SBLK < RPW:
            ns = j // SBLK + 1
            stage_pending = [
                pltpu.async_copy(src_hbm.at[pl.ds(r0 + ns * SBLK, SBLK)],
                                 src_v.at[ns % 2], sem_i),
                pltpu.async_copy(dst_hbm.at[pl.ds(r0 + ns * SBLK, SBLK)],
                                 dst_v.at[ns % 2], sem_i),
            ]
        if j >= 1:
            scat[j - 1].wait()
        if j + 1 < RPW:
            nsp = ((j + 1) // SBLK) % 2
            nb = (j + 1) % SBLK
            if nb == 0:
                for d in stage_pending:
                    d.wait()
                stage_pending = []
            gather[j + 1] = pltpu.async_copy(
                table_hbm.at[src_v.at[nsp, nb]], buf_v.at[(j + 1) % 2], sem_g)
        gather[j].wait()
        scat[j] = pltpu.async_copy(buf_v.at[bp], acc_sh.at[dst_v.at[sp, b]],
                                   sem_s, add=True)
        if extra is not None:
            ones_v, deg_sh, sem_d = extra
            degs.append(pltpu.async_copy(ones_v, deg_sh.at[dst_v.at[sp, b]],
                                         sem_d, add=True))
    scat[RPW - 1].wait()
    for d in degs:
        d.wait()


@functools.partial(
    pl.kernel,
    mesh=_mesh,
    out_type=(
        jax.ShapeDtypeStruct((NC, NPAD, D), jnp.float32),
        jax.ShapeDtypeStruct((NC, NPAD), jnp.float32),
    ),
    scratch_types=[
        pltpu.VMEM((2, SBLK, 128), jnp.int32),
        pltpu.VMEM((2, SBLK, 128), jnp.int32),
        pltpu.VMEM((2, 128, D), jnp.float32),
        pltpu.VMEM((128,), jnp.float32),
        pltpu.VMEM((RPS,), jnp.float32),
        pltpu.SemaphoreType.DMA,
        pltpu.SemaphoreType.DMA,
        pltpu.SemaphoreType.DMA,
        pltpu.SemaphoreType.DMA,
        pltpu.VMEM_SHARED((NPAD, D), jnp.float32),
        pltpu.VMEM_SHARED((NPAD,), jnp.float32),
    ],
    compiler_params=_sc_params,
)
def _sc_layer1_agg(x_hbm, src_hbm, dst_hbm, p_out, deg_out,
                   src_v, dst_v, rows_v, ones_v, zd_v,
                   sem_i, sem_g, sem_s, sem_d, acc_sh, deg_sh):
    c = lax.axis_index("c")
    s = lax.axis_index("s")
    wid = s * NC + c
    r0 = wid * RPW

    _fill_rows(rows_v.at[0], 128, D, 0.0)
    _fill_flat(zd_v, RPS, 0.0)
    _fill_flat(ones_v, 128, 1.0)
    for k in range(RPS // 128):
        pltpu.sync_copy(rows_v.at[0], acc_sh.at[pl.ds(s * RPS + k * 128, 128)])
    pltpu.sync_copy(zd_v, deg_sh.at[pl.ds(s * RPS, RPS)])
    plsc.subcore_barrier()

    _sc_agg_pipeline(x_hbm, src_hbm, dst_hbm, src_v, dst_v, rows_v,
                     sem_i, sem_g, sem_s, r0, acc_sh,
                     extra=(ones_v, deg_sh, sem_d))
    plsc.subcore_barrier()

    for k in range(RPS // 128):
        sl = pl.ds(s * RPS + k * 128, 128)
        pltpu.sync_copy(acc_sh.at[sl], rows_v.at[0])
        pltpu.sync_copy(rows_v.at[0], p_out.at[c, sl])
    pltpu.sync_copy(deg_sh.at[pl.ds(s * RPS, RPS)], zd_v)
    pltpu.sync_copy(zd_v, deg_out.at[c, pl.ds(s * RPS, RPS)])


@functools.partial(
    pl.kernel,
    mesh=_mesh,
    out_type=jax.ShapeDtypeStruct((NC, NPAD, L), jnp.float32),
    scratch_types=[
        pltpu.VMEM((2, SBLK, 128), jnp.int32),
        pltpu.VMEM((2, SBLK, 128), jnp.int32),
        pltpu.VMEM((2, 128, L), jnp.float32),
        pltpu.VMEM((128, L), jnp.float32),
        pltpu.SemaphoreType.DMA,
        pltpu.SemaphoreType.DMA,
        pltpu.SemaphoreType.DMA,
        pltpu.VMEM_SHARED((NPAD, L), jnp.float32),
    ],
    compiler_params=_sc_params,
)
def _sc_layer2_agg(s16_hbm, src_hbm, dst_hbm, s_out,
                   src_v, dst_v, vals_v, zd_v, sem_i, sem_g, sem_s, acc_sh):
    c = lax.axis_index("c")
    s = lax.axis_index("s")
    wid = s * NC + c
    r0 = wid * RPW

    _fill_rows(zd_v, 128, L, 0.0)
    for k in range(RPS // 128):
        pltpu.sync_copy(zd_v, acc_sh.at[pl.ds(s * RPS + k * 128, 128)])
    plsc.subcore_barrier()

    _sc_agg_pipeline(s16_hbm, src_hbm, dst_hbm, src_v, dst_v, vals_v,
                     sem_i, sem_g, sem_s, r0, acc_sh)
    plsc.subcore_barrier()

    for k in range(RPS // 128):
        sl = pl.ds(s * RPS + k * 128, 128)
        pltpu.sync_copy(acc_sh.at[sl], zd_v)
        pltpu.sync_copy(zd_v, s_out.at[c, sl])


_RB = 1024


def _tc_layer_body(x_ref, p0_ref, p1_ref, d0_ref, d1_ref,
                   ws1_ref, wn1_ref, b1_ref, ws2_ref, wn2_ref, b2_ref,
                   s_ref, t_ref):
    d = jnp.maximum(d0_ref[...] + d1_ref[...], 1.0)
    agg = (p0_ref[...] + p1_ref[...]) / d
    h = x_ref[...] @ ws1_ref[...] + agg @ wn1_ref[...] + b1_ref[...]
    h = jax.nn.sigmoid(h)
    s_ref[...] = jnp.broadcast_to(h @ wn2_ref[...], (_RB, L))
    t_ref[...] = h @ ws2_ref[...] + b2_ref[...]


def _tc_layer(x, p0, p1, d0, d1, ws1, wn1, b1, ws2, wn2, b2):
    grid = (NPAD // _RB,)
    row = lambda i: (i, 0)
    full = lambda i: (0, 0)
    return pl.pallas_call(
        _tc_layer_body,
        grid=grid,
        in_specs=[
            pl.BlockSpec((_RB, D), row),
            pl.BlockSpec((_RB, D), row),
            pl.BlockSpec((_RB, D), row),
            pl.BlockSpec((_RB, 1), row),
            pl.BlockSpec((_RB, 1), row),
            pl.BlockSpec((D, D), full),
            pl.BlockSpec((D, D), full),
            pl.BlockSpec((1, D), full),
            pl.BlockSpec((D, 1), full),
            pl.BlockSpec((D, 1), full),
            pl.BlockSpec((1, 1), full),
        ],
        out_specs=[
            pl.BlockSpec((_RB, L), row),
            pl.BlockSpec((_RB, 1), row),
        ],
        out_shape=[
            jax.ShapeDtypeStruct((NPAD, L), jnp.float32),
            jax.ShapeDtypeStruct((NPAD, 1), jnp.float32),
        ],
    )(x, p0, p1, d0, d1, ws1, wn1, b1, ws2, wn2, b2)


def _tc_final_body(t_ref, s0_ref, s1_ref, d0_ref, d1_ref, o_ref):
    d = jnp.maximum(d0_ref[...] + d1_ref[...], 1.0)
    o_ref[...] = t_ref[...] + (s0_ref[:, 0:1] + s1_ref[:, 0:1]) / d


def _tc_final(t, s0, s1, d0, d1):
    grid = (NPAD // _RB,)
    row = lambda i: (i, 0)
    return pl.pallas_call(
        _tc_final_body,
        grid=grid,
        in_specs=[
            pl.BlockSpec((_RB, 1), row),
            pl.BlockSpec((_RB, L), row),
            pl.BlockSpec((_RB, L), row),
            pl.BlockSpec((_RB, 1), row),
            pl.BlockSpec((_RB, 1), row),
        ],
        out_specs=pl.BlockSpec((_RB, 1), row),
        out_shape=jax.ShapeDtypeStruct((NPAD, 1), jnp.float32),
    )(t, s0, s1, d0, d1)


def kernel(x, edge_index, W_self1, W_neigh1, b1, W_self2, W_neigh2, b2):
    x_pad = jnp.concatenate([x, jnp.zeros((NPAD - N, D), jnp.float32)])
    pad_i = jnp.arange(EPAD, dtype=jnp.int32)
    src2d = jnp.concatenate([edge_index[0], pad_i % 128]).reshape(EROWS, 128)
    dst2d = jnp.concatenate([edge_index[1], N + pad_i % (NPAD - N)]).reshape(EROWS, 128)

    p, deg = _sc_layer1_agg(x_pad, src2d, dst2d)
    d0 = deg[0].reshape(NPAD, 1)
    d1 = deg[1].reshape(NPAD, 1)
    s16, t = _tc_layer(x_pad, p[0], p[1], d0, d1,
                       W_self1, W_neigh1, b1.reshape(1, D),
                       W_self2, W_neigh2, b2.reshape(1, 1))
    s2 = _sc_layer2_agg(s16, src2d, dst2d)
    out = _tc_final(t, s2[0], s2[1], d0, d1)
    return out[:N]
